# baseline (speedup 1.0000x reference)
import functools

import jax
import jax.numpy as jnp
from jax import lax
from jax.experimental import pallas as pl
from jax.experimental.pallas import tpu as pltpu
from jax.experimental.pallas import tpu_sc as plsc

_B = 2
_C = 10000
_F = 2
_H = 128
_N = 50000
_GB = 120
_SB = 2 * _GB
_NBLK = (_N + _SB - 1) // _SB
_NW = 32
_FULL = _NBLK // _NW
_EXTRA_W = _NBLK - _FULL * _NW

_mesh = plsc.VectorSubcoreMesh(core_axis_name="c", subcore_axis_name="s")


@functools.partial(
    pl.kernel,
    mesh=_mesh,
    out_type=jax.ShapeDtypeStruct((_B, _N, _F, _H), jnp.float32),
    scratch_types=[
        pltpu.VMEM((_FULL + 1, _GB), jnp.int32),
        pltpu.VMEM((_FULL + 1, _GB), jnp.int32),
        pltpu.VMEM((_SB, _F, _H), jnp.float32),
        pltpu.VMEM((_SB, _F, _H), jnp.float32),
        pltpu.SemaphoreType.DMA,
        pltpu.SemaphoreType.DMA,
        pltpu.SemaphoreType.DMA,
        pltpu.SemaphoreType.DMA,
        pltpu.SemaphoreType.DMA,
    ],
)
def _unpool(x_hbm, idx_hbm, out_hbm, idx0, idx1, buf0, buf1,
            isem, gs0, gs1, ss0, ss1):
    idxs = (idx0, idx1)
    bufs = (buf0, buf1)
    gsems = (gs0, gs1)
    ssems = (ss0, ss1)
    wid = lax.axis_index("s") * 2 + lax.axis_index("c")

    start = wid * _FULL + jnp.minimum(wid, _EXTRA_W)

    def base_of(p):
        base = jnp.minimum((start + p) * _SB, _N - _SB)
        return pl.multiple_of(base, 8)

    icopies = [
        pltpu.async_copy(
            idx_hbm.at[pl.ds(base_of(p) + g * _GB, _GB)], idxs[g].at[p], isem)
        for p in range(_FULL + 1)
        for g in range(2)
    ]
    for c in icopies:
        c.wait()

    def gpair(p, b):
        buf = bufs[b]
        sem = gsems[b]
        g0 = pltpu.async_copy(
            x_hbm.at[b].at[idx0.at[p]], buf.at[pl.ds(0, _GB)], sem)
        g1 = pltpu.async_copy(
            x_hbm.at[b].at[idx1.at[p]], buf.at[pl.ds(_GB, _GB)], sem)
        return g0, g1

    def spush(p, b):
        return pltpu.async_copy(
            bufs[b], out_hbm.at[b, pl.ds(base_of(p), _SB)], ssems[b])

    def drain_s(b):
        pltpu.make_async_copy(
            x_hbm.at[0, pl.ds(0, _SB)], bufs[b], ssems[b]).wait()

    def drain_g(b):
        pltpu.make_async_copy(
            x_hbm.at[0, pl.ds(0, _GB)], bufs[b].at[pl.ds(0, _GB)],
            gsems[b]).wait()

    p0 = gpair(0, 0)
    gpair(0, 1)
    p0[0].wait()
    p0[1].wait()
    spush(0, 0)

    def body(p, carry):
        drain_s(0)
        gpair(p, 0)
        drain_g(1)
        drain_g(1)
        spush(p - 1, 1)
        drain_s(1)
        gpair(p, 1)
        drain_g(0)
        drain_g(0)
        spush(p, 0)
        return carry

    lax.fori_loop(1, _FULL, body, 0)

    drain_g(1)
    drain_g(1)
    spush(_FULL - 1, 1)
    drain_s(0)
    drain_s(1)

    @pl.when(wid < _EXTRA_W)
    def _():
        for b in range(_B):
            g0, g1 = gpair(_FULL, b)
            g0.wait()
            g1.wait()
            spush(_FULL, b).wait()


def kernel(x, hierarchy_mapping, num_fine_nodes):
    idx = hierarchy_mapping.astype(jnp.int32)
    return _unpool(x, idx)


# speedup vs baseline: 1.0587x; 1.0587x over previous
import functools

import jax
import jax.numpy as jnp
from jax import lax
from jax.experimental import pallas as pl
from jax.experimental.pallas import tpu as pltpu
from jax.experimental.pallas import tpu_sc as plsc

_B = 2
_C = 10000
_F = 2
_H = 128
_N = 50000
_GB = 112
_SB = 2 * _GB
_NBLK = (_N + _SB - 1) // _SB
_NW = 32
_FULL = _NBLK // _NW

_mesh = plsc.VectorSubcoreMesh(core_axis_name="c", subcore_axis_name="s")


@functools.partial(
    pl.kernel,
    mesh=_mesh,
    out_type=jax.ShapeDtypeStruct((_B, _N, _F, _H), jnp.float32),
    scratch_types=[
        pltpu.VMEM((_FULL, _GB), jnp.int32),
        pltpu.VMEM((_FULL, _GB), jnp.int32),
        pltpu.VMEM((_SB, _F, _H), jnp.float32),
        pltpu.VMEM((_SB, _F, _H), jnp.float32),
        pltpu.SemaphoreType.DMA,
        pltpu.SemaphoreType.DMA,
        pltpu.SemaphoreType.DMA,
        pltpu.SemaphoreType.DMA,
        pltpu.SemaphoreType.DMA,
    ],
)
def _unpool(x_hbm, idx_hbm, out_hbm, idx0, idx1, buf0, buf1,
            isem, gs0, gs1, ss0, ss1):
    idxs = (idx0, idx1)
    bufs = (buf0, buf1)
    gsems = (gs0, gs1)
    ssems = (ss0, ss1)
    wid = lax.axis_index("s") * 2 + lax.axis_index("c")

    start = wid * _FULL

    def base_of(p):
        base = jnp.minimum((start + p) * _SB, _N - _SB)
        return pl.multiple_of(base, 8)

    icopies = [
        pltpu.async_copy(
            idx_hbm.at[pl.ds(base_of(p) + g * _GB, _GB)], idxs[g].at[p], isem)
        for p in range(_FULL)
        for g in range(2)
    ]
    for c in icopies:
        c.wait()

    def gpair(p, b):
        buf = bufs[b]
        sem = gsems[b]
        g0 = pltpu.async_copy(
            x_hbm.at[b].at[idx0.at[p]], buf.at[pl.ds(0, _GB)], sem)
        g1 = pltpu.async_copy(
            x_hbm.at[b].at[idx1.at[p]], buf.at[pl.ds(_GB, _GB)], sem)
        return g0, g1

    def spush(p, b):
        return pltpu.async_copy(
            bufs[b], out_hbm.at[b, pl.ds(base_of(p), _SB)], ssems[b])

    def drain_s(b):
        pltpu.make_async_copy(
            x_hbm.at[0, pl.ds(0, _SB)], bufs[b], ssems[b]).wait()

    def drain_g(b):
        pltpu.make_async_copy(
            x_hbm.at[0, pl.ds(0, _GB)], bufs[b].at[pl.ds(0, _GB)],
            gsems[b]).wait()

    p0 = gpair(0, 0)
    gpair(0, 1)
    p0[0].wait()
    p0[1].wait()
    spush(0, 0)

    def body(p, carry):
        drain_s(0)
        gpair(p, 0)
        drain_g(1)
        drain_g(1)
        spush(p - 1, 1)
        drain_s(1)
        gpair(p, 1)
        drain_g(0)
        drain_g(0)
        spush(p, 0)
        return carry

    lax.fori_loop(1, _FULL, body, 0)

    drain_g(1)
    drain_g(1)
    spush(_FULL - 1, 1)
    drain_s(0)
    drain_s(1)


def kernel(x, hierarchy_mapping, num_fine_nodes):
    idx = hierarchy_mapping.astype(jnp.int32)
    return _unpool(x, idx)
